# baseline (speedup 1.0000x reference)
import functools

import jax
import jax.numpy as jnp
from jax.experimental import pallas as pl
from jax.experimental.pallas import tpu as pltpu
from jax.experimental.pallas import tpu_sc as plsc

_LANES = 16


@jax.jit
def kernel(x, token_table, pos_table):
    B, L = x.shape
    V, D = token_table.shape
    N = B * L
    W = 8 * L

    tok_lin = (
        token_table.reshape(V // 32, 4, 8, D)
        .transpose(0, 2, 1, 3)
        .reshape(V, D)
    )
    xi = x.reshape(N).astype(jnp.int32)
    x_flat = (xi & ~jnp.int32(31)) | ((xi & 7) << 2) | ((xi >> 3) & 3)

    @functools.partial(
        pl.kernel,
        out_type=jax.ShapeDtypeStruct((N, D), jnp.float32),
        mesh=plsc.VectorSubcoreMesh(
            core_axis_name="core", subcore_axis_name="subcore"
        ),
        compiler_params=pltpu.CompilerParams(use_tc_tiling_on_sc=False),
    )
    def sc_embed(tok_hbm, idx_hbm, out_hbm):
        def body(i_vmem, o_vmem):
            pltpu.sync_copy(tok_hbm.at[i_vmem], o_vmem)

        pltpu.emit_pipeline(
            body,
            grid=(N // W,),
            in_specs=[pl.BlockSpec((W,), lambda i: (i,))],
            out_specs=[pl.BlockSpec((W, D), lambda i: (i, 0))],
            core_axis_name=("core", "subcore"),
            dimension_semantics=(pltpu.PARALLEL,),
        )(idx_hbm, out_hbm)

    flat = sc_embed(tok_lin, x_flat)

    G = L // 4
    t2 = flat.reshape(N // 4, 128)
    BB = 256

    pos128 = pos_table.reshape(G, 128)

    def tc_body(in_ref, pos_ref, out_ref):
        v = in_ref[...].reshape(BB, G, 128)
        for g in range(G):
            out_ref[g] = v[:, g, :].T + pos_ref[g][:, None]

    out128 = pl.pallas_call(
        tc_body,
        grid=(B // BB,),
        in_specs=[
            pl.BlockSpec((G * BB, 128), lambda i: (i, 0)),
            pl.BlockSpec((G, 128), lambda i: (0, 0)),
        ],
        out_specs=pl.BlockSpec((G, 128, BB), lambda i: (0, 0, i)),
        out_shape=jax.ShapeDtypeStruct((G, 128, B), jnp.float32),
    )(t2, pos128)
    return out128.reshape(L, D, B).transpose(2, 0, 1)
